# baseline (speedup 1.0000x reference)
import functools

import jax
import jax.numpy as jnp
from jax import lax
from jax.experimental import pallas as pl
from jax.experimental.pallas import tpu as pltpu
from jax.experimental.pallas import tpu_sc as plsc

TOPK = 8
N_EXPERTS = 64
D_MODEL = 2048
N_ROWS = 16384
BM = 2048
NC, NS, L = 2, 16, 16
NW = NC * NS
ROWS_PER_W = N_ROWS // NW
GROUPS_PER_W = ROWS_PER_W // L


def _logits_block(x_ref, w_ref, out_ref):
    w = jax.nn.sigmoid(w_ref[...])
    out_ref[...] = jax.lax.dot_general(
        w, x_ref[...],
        dimension_numbers=(((1,), (1,)), ((), ())),
        preferred_element_type=jnp.float32,
    )


def _tc_logits_t(x, weight_raw):
    return pl.pallas_call(
        _logits_block,
        grid=(N_ROWS // BM,),
        in_specs=[
            pl.BlockSpec((BM, D_MODEL), lambda i: (i, 0)),
            pl.BlockSpec((N_EXPERTS, D_MODEL), lambda i: (0, 0)),
        ],
        out_specs=pl.BlockSpec((N_EXPERTS, BM), lambda i: (0, i)),
        out_shape=jax.ShapeDtypeStruct((N_EXPERTS, N_ROWS), jnp.float32),
        compiler_params=pltpu.CompilerParams(
            dimension_semantics=("arbitrary",),
        ),
    )(x, weight_raw)


def _sc_topk_body(lt_hbm, probs_hbm, idx_hbm, blk_v, pout_v, iout_v, sem):
    wid = lax.axis_index("s") * NC + lax.axis_index("c")
    base = wid * ROWS_PER_W
    pltpu.sync_copy(lt_hbm.at[:, pl.ds(base, ROWS_PER_W)], blk_v)

    def group(g, carry):
        g16 = g * L
        neg_inf = jnp.full((L,), -jnp.inf, dtype=jnp.float32)
        s = [neg_inf] * TOPK
        si = [jnp.zeros((L,), dtype=jnp.int32)] * TOPK
        for e in range(N_EXPERTS):
            v = blk_v[e, pl.ds(g16, L)]
            ei = jnp.full((L,), e, dtype=jnp.int32)
            c = [v > s[j] for j in range(TOPK)]
            ns = [None] * TOPK
            ni = [None] * TOPK
            for j in range(TOPK):
                if j == 0:
                    inner_v, inner_i = v, ei
                else:
                    inner_v = jnp.where(c[j - 1], s[j - 1], v)
                    inner_i = jnp.where(c[j - 1], si[j - 1], ei)
                ns[j] = jnp.where(c[j], inner_v, s[j])
                ni[j] = jnp.where(c[j], inner_i, si[j])
            s, si = ns, ni
        es = [jnp.exp(s[j] - s[0]) for j in range(TOPK)]
        tot = es[0]
        for j in range(1, TOPK):
            tot = tot + es[j]
        for j in range(TOPK):
            pout_v[j, pl.ds(g16, L)] = es[j] / tot
            iout_v[j, pl.ds(g16, L)] = si[j]
        return carry

    lax.fori_loop(0, GROUPS_PER_W, group, 0)

    pltpu.sync_copy(pout_v, probs_hbm.at[:, pl.ds(base, ROWS_PER_W)])
    pltpu.sync_copy(iout_v, idx_hbm.at[:, pl.ds(base, ROWS_PER_W)])


def _sc_topk(logits_t):
    mesh = plsc.VectorSubcoreMesh(core_axis_name="c", subcore_axis_name="s")
    f = functools.partial(
        pl.kernel,
        mesh=mesh,
        out_type=[
            jax.ShapeDtypeStruct((TOPK, N_ROWS), jnp.float32),
            jax.ShapeDtypeStruct((TOPK, N_ROWS), jnp.int32),
        ],
        scratch_types=[
            pltpu.VMEM((N_EXPERTS, ROWS_PER_W), jnp.float32),
            pltpu.VMEM((TOPK, ROWS_PER_W), jnp.float32),
            pltpu.VMEM((TOPK, ROWS_PER_W), jnp.int32),
            pltpu.SemaphoreType.DMA,
        ],
    )(_sc_topk_body)
    return f(logits_t)


@jax.jit
def kernel(x, weight_raw):
    logits_t = _tc_logits_t(x, weight_raw)
    probs_t, idx_t = _sc_topk(logits_t)
    return (probs_t.T, idx_t.T)
